# Initial kernel scaffold; baseline (speedup 1.0000x reference)
import jax
import jax.numpy as jnp
from jax.experimental import pallas as pl


def kernel(x, edge_index, batch, energy, mode, W1, b1, gcn_bn_g, gcn_bn_b, W2, b2, bn_g, bn_b, fc1_W, fc1_b, bn1_g, bn1_b, fc3_W, fc3_b):
    raise NotImplementedError("write your pallas kernel here")


# speedup vs baseline: 15.3337x; 15.3337x over previous
import functools

import jax
import jax.numpy as jnp
from jax import lax
from jax.experimental import pallas as pl
from jax.experimental.pallas import tpu as pltpu
from jax.experimental.pallas import tpu_sc as plsc

_N = 10000
_E = 320000
_D = 128
_G = 64
_N2 = 10240

_NC = 2
_NS = 16
_NW = _NC * _NS
_EPT = _E // _NW
_EK = 400
_NWIN = _EPT // _EK
_RPT = _N2 // _NS
_ZR = 64

_BR = 512
_NBLK = _N2 // _BR


def _mesh():
    return plsc.VectorSubcoreMesh(
        core_axis_name="c", subcore_axis_name="s",
        num_cores=_NC, num_subcores=_NS)


def _deg_body(dst_hbm, zeros_hbm, out_hbm, dstv, onesv, bufv, acc_sh, sem):
    c = lax.axis_index("c")
    s = lax.axis_index("s")
    wid = s * _NC + c

    def fill_ones(i, carry):
        onesv[pl.ds(i * 16, 16)] = jnp.ones((16,), jnp.float32)
        return carry
    lax.fori_loop(0, _EK // 16, fill_ones, 0)

    pltpu.sync_copy(zeros_hbm.at[pl.ds(0, _RPT)], bufv)
    pltpu.sync_copy(bufv, acc_sh.at[pl.ds(s * _RPT, _RPT)])
    plsc.subcore_barrier()

    def window(i, carry):
        base = wid * _EPT + i * _EK
        pltpu.sync_copy(dst_hbm.at[pl.ds(base, _EK)], dstv)
        pltpu.sync_copy(onesv, acc_sh.at[dstv], add=True)
        return carry
    lax.fori_loop(0, _NWIN, window, 0)
    plsc.subcore_barrier()

    pltpu.sync_copy(acc_sh.at[pl.ds(s * _RPT, _RPT)], bufv)
    pltpu.sync_copy(bufv, out_hbm.at[pl.ds(c * _N2 + s * _RPT, _RPT)])


def _sc_degree(dst, zeros_row):
    return pl.kernel(
        _deg_body,
        out_type=jax.ShapeDtypeStruct((_NC * _N2,), jnp.float32),
        mesh=_mesh(),
        scratch_types=[
            pltpu.VMEM((_EK,), jnp.int32),
            pltpu.VMEM((_EK,), jnp.float32),
            pltpu.VMEM((_RPT,), jnp.float32),
            pltpu.VMEM_SHARED((_N2,), jnp.float32),
            pltpu.SemaphoreType.DMA,
        ],
    )(dst, zeros_row)


_DH = _D // 2


def _prop_body(hlo_hbm, hhi_hbm, src_hbm, dst_hbm, zeros_hbm, out_hbm,
               srcv, dstv, rows, bufv, acc_sh, sem):
    c = lax.axis_index("c")
    s = lax.axis_index("s")
    wid = s * _NC + c

    for p, hs_hbm in ((0, hlo_hbm), (1, hhi_hbm)):
        pltpu.sync_copy(zeros_hbm, bufv)
        for j in range(_RPT // _ZR):
            pltpu.sync_copy(bufv, acc_sh.at[pl.ds(s * _RPT + j * _ZR, _ZR)])
        plsc.subcore_barrier()

        def window(i, carry):
            base = wid * _EPT + i * _EK
            pltpu.sync_copy(src_hbm.at[pl.ds(base, _EK)], srcv)
            pltpu.sync_copy(dst_hbm.at[pl.ds(base, _EK)], dstv)
            pltpu.async_copy(hs_hbm.at[srcv], rows, sem).wait()
            pltpu.sync_copy(rows, acc_sh.at[dstv], add=True)
            return carry
        lax.fori_loop(0, _NWIN, window, 0)
        plsc.subcore_barrier()

        for j in range(_RPT // _ZR):
            pltpu.sync_copy(acc_sh.at[pl.ds(s * _RPT + j * _ZR, _ZR)], bufv)
            pltpu.sync_copy(
                bufv,
                out_hbm.at[pl.ds((p * _NC + c) * _N2 + s * _RPT + j * _ZR,
                                 _ZR)])
        plsc.subcore_barrier()


def _sc_propagate(hs_lo, hs_hi, src, dst, zeros_blk):
    return pl.kernel(
        _prop_body,
        out_type=jax.ShapeDtypeStruct((2 * _NC * _N2, _DH), jnp.float32),
        mesh=_mesh(),
        scratch_types=[
            pltpu.VMEM((_EK,), jnp.int32),
            pltpu.VMEM((_EK,), jnp.int32),
            pltpu.VMEM((_EK, _DH), jnp.float32),
            pltpu.VMEM((_ZR, _DH), jnp.float32),
            pltpu.VMEM_SHARED((_N2, _DH), jnp.float32),
            pltpu.SemaphoreType.DMA,
        ],
        compiler_params=pltpu.CompilerParams(use_tc_tiling_on_sc=False),
    )(hs_lo, hs_hi, src, dst, zeros_blk)


def _tc_a_body(x_ref, degt_ref, w_ref, h_ref, hlo_ref, hhi_ref, dinv_ref):
    d = degt_ref[:, 0:1] + degt_ref[:, 1:2] + 1.0
    dinv = 1.0 / jnp.sqrt(d)
    h = jnp.dot(x_ref[:], w_ref[:], preferred_element_type=jnp.float32)
    h_ref[:] = h
    hs = h * dinv
    hlo_ref[:] = hs[:, :_DH]
    hhi_ref[:] = hs[:, _DH:]
    dinv_ref[:] = dinv


def _tc_a(x2, degt, W1):
    return pl.pallas_call(
        _tc_a_body,
        grid=(_NBLK,),
        in_specs=[
            pl.BlockSpec((_BR, _D), lambda i: (i, 0)),
            pl.BlockSpec((_BR, _NC), lambda i: (i, 0)),
            pl.BlockSpec((_D, _D), lambda i: (0, 0)),
        ],
        out_specs=[
            pl.BlockSpec((_BR, _D), lambda i: (i, 0)),
            pl.BlockSpec((_BR, _DH), lambda i: (i, 0)),
            pl.BlockSpec((_BR, _DH), lambda i: (i, 0)),
            pl.BlockSpec((_BR, 1), lambda i: (i, 0)),
        ],
        out_shape=[
            jax.ShapeDtypeStruct((_N2, _D), jnp.float32),
            jax.ShapeDtypeStruct((_N2, _DH), jnp.float32),
            jax.ShapeDtypeStruct((_N2, _DH), jnp.float32),
            jax.ShapeDtypeStruct((_N2, 1), jnp.float32),
        ],
    )(x2, degt, W1)


def _ln_rows(y, g, b, eps=1e-5):
    mu = jnp.mean(y, axis=-1, keepdims=True)
    var = jnp.mean((y - mu) * (y - mu), axis=-1, keepdims=True)
    return (y - mu) / jnp.sqrt(var + eps) * g + b


def _acc_full(lo0_ref, lo1_ref, hi0_ref, hi1_ref):
    return jnp.concatenate(
        [lo0_ref[:] + lo1_ref[:], hi0_ref[:] + hi1_ref[:]], axis=1)


def _tc_b_body(lo0_ref, lo1_ref, hi0_ref, hi1_ref, h1_ref, dinv_ref, b1_ref,
               g_ref, bb_ref, w2_ref, h2_ref, hlo_ref, hhi_ref):
    dinv = dinv_ref[:]
    acc = _acc_full(lo0_ref, lo1_ref, hi0_ref, hi1_ref)
    y = dinv * acc + (dinv * dinv) * h1_ref[:] + b1_ref[:]
    y = jnp.maximum(y, 0.0)
    y = _ln_rows(y, g_ref[:], bb_ref[:])
    h2 = jnp.dot(y, w2_ref[:], preferred_element_type=jnp.float32)
    h2_ref[:] = h2
    hs = h2 * dinv
    hlo_ref[:] = hs[:, :_DH]
    hhi_ref[:] = hs[:, _DH:]


def _tc_b(lo0, lo1, hi0, hi1, h1, dinv, b1r, gr, br, W2):
    row = lambda i: (i, 0)
    fixed = lambda i: (0, 0)
    return pl.pallas_call(
        _tc_b_body,
        grid=(_NBLK,),
        in_specs=[
            pl.BlockSpec((_BR, _DH), row),
            pl.BlockSpec((_BR, _DH), row),
            pl.BlockSpec((_BR, _DH), row),
            pl.BlockSpec((_BR, _DH), row),
            pl.BlockSpec((_BR, _D), row),
            pl.BlockSpec((_BR, 1), row),
            pl.BlockSpec((1, _D), fixed),
            pl.BlockSpec((1, _D), fixed),
            pl.BlockSpec((1, _D), fixed),
            pl.BlockSpec((_D, _D), fixed),
        ],
        out_specs=[
            pl.BlockSpec((_BR, _D), row),
            pl.BlockSpec((_BR, _DH), row),
            pl.BlockSpec((_BR, _DH), row),
        ],
        out_shape=[
            jax.ShapeDtypeStruct((_N2, _D), jnp.float32),
            jax.ShapeDtypeStruct((_N2, _DH), jnp.float32),
            jax.ShapeDtypeStruct((_N2, _DH), jnp.float32),
        ],
    )(lo0, lo1, hi0, hi1, h1, dinv, b1r, gr, br, W2)


def _tc_c_body(lo0_ref, lo1_ref, hi0_ref, hi1_ref, h2_ref, dinv_ref, b2_ref,
               batch_ref, bng_ref, bnb_ref, fc1w_ref, fc1b_ref, bn1g_ref,
               bn1b_ref, fc3w_ref, fc3b_ref, out_ref, psum, pcnt):
    i = pl.program_id(0)

    @pl.when(i == 0)
    def _init():
        psum[:] = jnp.zeros_like(psum)
        pcnt[:] = jnp.zeros_like(pcnt)

    dinv = dinv_ref[:]
    acc = _acc_full(lo0_ref, lo1_ref, hi0_ref, hi1_ref)
    y = dinv * acc + (dinv * dinv) * h2_ref[:] + b2_ref[:]
    y = jnp.maximum(y, 0.0)

    ids = batch_ref[:]
    iota = lax.broadcasted_iota(jnp.int32, (_BR, _D), 1)
    oh = (iota == ids).astype(jnp.float32)
    dn = (((0,), (0,)), ((), ()))
    psum[:] += lax.dot_general(oh, y, dimension_numbers=dn,
                               preferred_element_type=jnp.float32,
                               precision=lax.Precision.HIGHEST)
    pcnt[:] += lax.dot_general(oh, jnp.ones((_BR, 1), jnp.float32),
                               dimension_numbers=dn,
                               preferred_element_type=jnp.float32,
                               precision=lax.Precision.HIGHEST)

    @pl.when(i == _NBLK - 1)
    def _head():
        pooled = psum[:] / jnp.maximum(pcnt[:], 1.0)
        h = _ln_rows(pooled, bng_ref[:], bnb_ref[:])
        h = jnp.dot(h, fc1w_ref[:],
                    preferred_element_type=jnp.float32) + fc1b_ref[:]
        h = jnp.maximum(h, 0.0)
        h = _ln_rows(h, bn1g_ref[:], bn1b_ref[:])
        om = jnp.dot(h, fc3w_ref[:], preferred_element_type=jnp.float32)
        out_ref[:] = om[:, 0:1] + fc3b_ref[:]


def _tc_c(lo0, lo1, hi0, hi1, h2, dinv, b2r, batch2, bng, bnb, fc1W, fc1b,
          bn1g, bn1b, fc3wr, fc3br):
    row = lambda i: (i, 0)
    fixed = lambda i: (0, 0)
    return pl.pallas_call(
        _tc_c_body,
        grid=(_NBLK,),
        in_specs=[
            pl.BlockSpec((_BR, _DH), row),
            pl.BlockSpec((_BR, _DH), row),
            pl.BlockSpec((_BR, _DH), row),
            pl.BlockSpec((_BR, _DH), row),
            pl.BlockSpec((_BR, _D), row),
            pl.BlockSpec((_BR, 1), row),
            pl.BlockSpec((1, _D), fixed),
            pl.BlockSpec((_BR, 1), row),
            pl.BlockSpec((1, _D), fixed),
            pl.BlockSpec((1, _D), fixed),
            pl.BlockSpec((_D, _D), fixed),
            pl.BlockSpec((1, _D), fixed),
            pl.BlockSpec((1, _D), fixed),
            pl.BlockSpec((1, _D), fixed),
            pl.BlockSpec((_D, _D), fixed),
            pl.BlockSpec((1, 1), fixed),
        ],
        out_specs=pl.BlockSpec((_D, 1), fixed),
        out_shape=jax.ShapeDtypeStruct((_D, 1), jnp.float32),
        scratch_shapes=[
            pltpu.VMEM((_D, _D), jnp.float32),
            pltpu.VMEM((_D, 1), jnp.float32),
        ],
    )(lo0, lo1, hi0, hi1, h2, dinv, b2r, batch2, bng, bnb, fc1W, fc1b,
      bn1g, bn1b, fc3wr, fc3br)


def kernel(x, edge_index, batch, energy, mode, W1, b1, gcn_bn_g, gcn_bn_b,
           W2, b2, bn_g, bn_b, fc1_W, fc1_b, bn1_g, bn1_b, fc3_W, fc3_b):
    src = edge_index[0]
    dst = edge_index[1]

    x2 = jnp.pad(x, ((0, _N2 - _N), (0, 0)))
    batch2 = jnp.pad(batch, (0, _N2 - _N),
                     constant_values=_D - 1).reshape(_N2, 1)
    zeros_row = jnp.zeros((_RPT,), jnp.float32)
    zeros_blk = jnp.zeros((_ZR, _DH), jnp.float32)

    deg_parts = _sc_degree(dst, zeros_row)
    degt = deg_parts.reshape(_NC, _N2).T

    h1, hs1_lo, hs1_hi, dinv = _tc_a(x2, degt, W1)

    p1 = _sc_propagate(hs1_lo, hs1_hi, src, dst, zeros_blk)
    q1 = p1.reshape(2 * _NC, _N2, _DH)
    h2, hs2_lo, hs2_hi = _tc_b(q1[0], q1[1], q1[2], q1[3], h1, dinv,
                               b1.reshape(1, _D), gcn_bn_g.reshape(1, _D),
                               gcn_bn_b.reshape(1, _D), W2)

    p2 = _sc_propagate(hs2_lo, hs2_hi, src, dst, zeros_blk)
    q2 = p2.reshape(2 * _NC, _N2, _DH)
    fc3wr = jnp.pad(fc3_W, ((0, 0), (0, _D - 1)))
    fc3br = fc3_b.reshape(1, 1)
    out = _tc_c(q2[0], q2[1], q2[2], q2[3], h2, dinv, b2.reshape(1, _D),
                batch2, bn_g.reshape(1, _D), bn_b.reshape(1, _D), fc1_W,
                fc1_b.reshape(1, _D), bn1_g.reshape(1, _D),
                bn1_b.reshape(1, _D), fc3wr, fc3br)
    return out[:_G]


# speedup vs baseline: 19.3590x; 1.2625x over previous
import functools

import jax
import jax.numpy as jnp
from jax import lax
from jax.experimental import pallas as pl
from jax.experimental.pallas import tpu as pltpu
from jax.experimental.pallas import tpu_sc as plsc

_N = 10000
_E = 320000
_D = 128
_G = 64
_N2 = 10240

_NC = 2
_NS = 16
_NW = _NC * _NS
_EPT = _E // _NW
_EK = 400
_NWIN = _EPT // _EK
_RPT = _N2 // _NS
_ZR = 64

_BR = 512
_NBLK = _N2 // _BR


def _mesh():
    return plsc.VectorSubcoreMesh(
        core_axis_name="c", subcore_axis_name="s",
        num_cores=_NC, num_subcores=_NS)


def _deg_body(dst_hbm, zeros_hbm, out_hbm, dstv, onesv, bufv, acc_sh, sem):
    c = lax.axis_index("c")
    s = lax.axis_index("s")
    wid = s * _NC + c

    def fill_ones(i, carry):
        onesv[pl.ds(i * 16, 16)] = jnp.ones((16,), jnp.float32)
        return carry
    lax.fori_loop(0, _EK // 16, fill_ones, 0)

    pltpu.sync_copy(zeros_hbm.at[pl.ds(0, _RPT)], bufv)
    pltpu.sync_copy(bufv, acc_sh.at[pl.ds(s * _RPT, _RPT)])
    plsc.subcore_barrier()

    def window(i, carry):
        base = wid * _EPT + i * _EK
        pltpu.sync_copy(dst_hbm.at[pl.ds(base, _EK)], dstv)
        pltpu.sync_copy(onesv, acc_sh.at[dstv], add=True)
        return carry
    lax.fori_loop(0, _NWIN, window, 0)
    plsc.subcore_barrier()

    pltpu.sync_copy(acc_sh.at[pl.ds(s * _RPT, _RPT)], bufv)
    pltpu.sync_copy(bufv, out_hbm.at[pl.ds(c * _N2 + s * _RPT, _RPT)])


def _sc_degree(dst, zeros_row):
    return pl.kernel(
        _deg_body,
        out_type=jax.ShapeDtypeStruct((_NC * _N2,), jnp.float32),
        mesh=_mesh(),
        scratch_types=[
            pltpu.VMEM((_EK,), jnp.int32),
            pltpu.VMEM((_EK,), jnp.float32),
            pltpu.VMEM((_RPT,), jnp.float32),
            pltpu.VMEM_SHARED((_N2,), jnp.float32),
            pltpu.SemaphoreType.DMA,
        ],
    )(dst, zeros_row)


_DH = _D // 2


def _prop_body(hlo_hbm, hhi_hbm, src_hbm, dst_hbm, zeros_hbm, out_hbm,
               srcv0, dstv0, rows0, srcv1, dstv1, rows1, bufv, acc_sh,
               sem0, sem1):
    c = lax.axis_index("c")
    s = lax.axis_index("s")
    wid = s * _NC + c
    bufs = ((srcv0, dstv0, rows0, sem0), (srcv1, dstv1, rows1, sem1))

    def fire(i, b):
        srcv, dstv, rows, sem = b
        base = wid * _EPT + i * _EK
        pltpu.sync_copy(src_hbm.at[pl.ds(base, _EK)], srcv)
        pltpu.sync_copy(dst_hbm.at[pl.ds(base, _EK)], dstv)
        return pltpu.async_copy(hs_hbm.at[srcv], rows, sem)

    for p, hs_hbm in ((0, hlo_hbm), (1, hhi_hbm)):
        pltpu.sync_copy(zeros_hbm, bufv)
        for j in range(_RPT // _ZR):
            pltpu.sync_copy(bufv, acc_sh.at[pl.ds(s * _RPT + j * _ZR, _ZR)])
        plsc.subcore_barrier()

        desc = fire(0, bufs[0])
        for i in range(_NWIN):
            cur = bufs[i % 2]
            if i + 1 < _NWIN:
                nxt_desc = fire(i + 1, bufs[(i + 1) % 2])
            desc.wait()
            pltpu.sync_copy(cur[2], acc_sh.at[cur[1]], add=True)
            if i + 1 < _NWIN:
                desc = nxt_desc
        plsc.subcore_barrier()

        for j in range(_RPT // _ZR):
            pltpu.sync_copy(acc_sh.at[pl.ds(s * _RPT + j * _ZR, _ZR)], bufv)
            pltpu.sync_copy(
                bufv,
                out_hbm.at[pl.ds((p * _NC + c) * _N2 + s * _RPT + j * _ZR,
                                 _ZR)])
        plsc.subcore_barrier()


def _sc_propagate(hs_lo, hs_hi, src, dst, zeros_blk):
    return pl.kernel(
        _prop_body,
        out_type=jax.ShapeDtypeStruct((2 * _NC * _N2, _DH), jnp.float32),
        mesh=_mesh(),
        scratch_types=[
            pltpu.VMEM((_EK,), jnp.int32),
            pltpu.VMEM((_EK,), jnp.int32),
            pltpu.VMEM((_EK, _DH), jnp.float32),
            pltpu.VMEM((_EK,), jnp.int32),
            pltpu.VMEM((_EK,), jnp.int32),
            pltpu.VMEM((_EK, _DH), jnp.float32),
            pltpu.VMEM((_ZR, _DH), jnp.float32),
            pltpu.VMEM_SHARED((_N2, _DH), jnp.float32),
            pltpu.SemaphoreType.DMA,
            pltpu.SemaphoreType.DMA,
        ],
        compiler_params=pltpu.CompilerParams(use_tc_tiling_on_sc=False),
    )(hs_lo, hs_hi, src, dst, zeros_blk)


def _tc_a_body(x_ref, degt_ref, w_ref, h_ref, hlo_ref, hhi_ref, dinv_ref):
    d = degt_ref[:, 0:1] + degt_ref[:, 1:2] + 1.0
    dinv = 1.0 / jnp.sqrt(d)
    h = jnp.dot(x_ref[:], w_ref[:], preferred_element_type=jnp.float32)
    h_ref[:] = h
    hs = h * dinv
    hlo_ref[:] = hs[:, :_DH]
    hhi_ref[:] = hs[:, _DH:]
    dinv_ref[:] = dinv


def _tc_a(x2, degt, W1):
    return pl.pallas_call(
        _tc_a_body,
        grid=(_NBLK,),
        in_specs=[
            pl.BlockSpec((_BR, _D), lambda i: (i, 0)),
            pl.BlockSpec((_BR, _NC), lambda i: (i, 0)),
            pl.BlockSpec((_D, _D), lambda i: (0, 0)),
        ],
        out_specs=[
            pl.BlockSpec((_BR, _D), lambda i: (i, 0)),
            pl.BlockSpec((_BR, _DH), lambda i: (i, 0)),
            pl.BlockSpec((_BR, _DH), lambda i: (i, 0)),
            pl.BlockSpec((_BR, 1), lambda i: (i, 0)),
        ],
        out_shape=[
            jax.ShapeDtypeStruct((_N2, _D), jnp.float32),
            jax.ShapeDtypeStruct((_N2, _DH), jnp.float32),
            jax.ShapeDtypeStruct((_N2, _DH), jnp.float32),
            jax.ShapeDtypeStruct((_N2, 1), jnp.float32),
        ],
    )(x2, degt, W1)


def _ln_rows(y, g, b, eps=1e-5):
    mu = jnp.mean(y, axis=-1, keepdims=True)
    var = jnp.mean((y - mu) * (y - mu), axis=-1, keepdims=True)
    return (y - mu) / jnp.sqrt(var + eps) * g + b


def _acc_full(lo0_ref, lo1_ref, hi0_ref, hi1_ref):
    return jnp.concatenate(
        [lo0_ref[:] + lo1_ref[:], hi0_ref[:] + hi1_ref[:]], axis=1)


def _tc_b_body(lo0_ref, lo1_ref, hi0_ref, hi1_ref, h1_ref, dinv_ref, b1_ref,
               g_ref, bb_ref, w2_ref, h2_ref, hlo_ref, hhi_ref):
    dinv = dinv_ref[:]
    acc = _acc_full(lo0_ref, lo1_ref, hi0_ref, hi1_ref)
    y = dinv * acc + (dinv * dinv) * h1_ref[:] + b1_ref[:]
    y = jnp.maximum(y, 0.0)
    y = _ln_rows(y, g_ref[:], bb_ref[:])
    h2 = jnp.dot(y, w2_ref[:], preferred_element_type=jnp.float32)
    h2_ref[:] = h2
    hs = h2 * dinv
    hlo_ref[:] = hs[:, :_DH]
    hhi_ref[:] = hs[:, _DH:]


def _tc_b(lo0, lo1, hi0, hi1, h1, dinv, b1r, gr, br, W2):
    row = lambda i: (i, 0)
    fixed = lambda i: (0, 0)
    return pl.pallas_call(
        _tc_b_body,
        grid=(_NBLK,),
        in_specs=[
            pl.BlockSpec((_BR, _DH), row),
            pl.BlockSpec((_BR, _DH), row),
            pl.BlockSpec((_BR, _DH), row),
            pl.BlockSpec((_BR, _DH), row),
            pl.BlockSpec((_BR, _D), row),
            pl.BlockSpec((_BR, 1), row),
            pl.BlockSpec((1, _D), fixed),
            pl.BlockSpec((1, _D), fixed),
            pl.BlockSpec((1, _D), fixed),
            pl.BlockSpec((_D, _D), fixed),
        ],
        out_specs=[
            pl.BlockSpec((_BR, _D), row),
            pl.BlockSpec((_BR, _DH), row),
            pl.BlockSpec((_BR, _DH), row),
        ],
        out_shape=[
            jax.ShapeDtypeStruct((_N2, _D), jnp.float32),
            jax.ShapeDtypeStruct((_N2, _DH), jnp.float32),
            jax.ShapeDtypeStruct((_N2, _DH), jnp.float32),
        ],
    )(lo0, lo1, hi0, hi1, h1, dinv, b1r, gr, br, W2)


def _tc_c_body(lo0_ref, lo1_ref, hi0_ref, hi1_ref, h2_ref, dinv_ref, b2_ref,
               batch_ref, bng_ref, bnb_ref, fc1w_ref, fc1b_ref, bn1g_ref,
               bn1b_ref, fc3w_ref, fc3b_ref, out_ref, psum, pcnt):
    i = pl.program_id(0)

    @pl.when(i == 0)
    def _init():
        psum[:] = jnp.zeros_like(psum)
        pcnt[:] = jnp.zeros_like(pcnt)

    dinv = dinv_ref[:]
    acc = _acc_full(lo0_ref, lo1_ref, hi0_ref, hi1_ref)
    y = dinv * acc + (dinv * dinv) * h2_ref[:] + b2_ref[:]
    y = jnp.maximum(y, 0.0)

    ids = batch_ref[:]
    iota = lax.broadcasted_iota(jnp.int32, (_BR, _D), 1)
    oh = (iota == ids).astype(jnp.float32)
    dn = (((0,), (0,)), ((), ()))
    psum[:] += lax.dot_general(oh, y, dimension_numbers=dn,
                               preferred_element_type=jnp.float32,
                               precision=lax.Precision.HIGHEST)
    pcnt[:] += lax.dot_general(oh, jnp.ones((_BR, 1), jnp.float32),
                               dimension_numbers=dn,
                               preferred_element_type=jnp.float32,
                               precision=lax.Precision.HIGHEST)

    @pl.when(i == _NBLK - 1)
    def _head():
        pooled = psum[:] / jnp.maximum(pcnt[:], 1.0)
        h = _ln_rows(pooled, bng_ref[:], bnb_ref[:])
        h = jnp.dot(h, fc1w_ref[:],
                    preferred_element_type=jnp.float32) + fc1b_ref[:]
        h = jnp.maximum(h, 0.0)
        h = _ln_rows(h, bn1g_ref[:], bn1b_ref[:])
        om = jnp.dot(h, fc3w_ref[:], preferred_element_type=jnp.float32)
        out_ref[:] = om[:, 0:1] + fc3b_ref[:]


def _tc_c(lo0, lo1, hi0, hi1, h2, dinv, b2r, batch2, bng, bnb, fc1W, fc1b,
          bn1g, bn1b, fc3wr, fc3br):
    row = lambda i: (i, 0)
    fixed = lambda i: (0, 0)
    return pl.pallas_call(
        _tc_c_body,
        grid=(_NBLK,),
        in_specs=[
            pl.BlockSpec((_BR, _DH), row),
            pl.BlockSpec((_BR, _DH), row),
            pl.BlockSpec((_BR, _DH), row),
            pl.BlockSpec((_BR, _DH), row),
            pl.BlockSpec((_BR, _D), row),
            pl.BlockSpec((_BR, 1), row),
            pl.BlockSpec((1, _D), fixed),
            pl.BlockSpec((_BR, 1), row),
            pl.BlockSpec((1, _D), fixed),
            pl.BlockSpec((1, _D), fixed),
            pl.BlockSpec((_D, _D), fixed),
            pl.BlockSpec((1, _D), fixed),
            pl.BlockSpec((1, _D), fixed),
            pl.BlockSpec((1, _D), fixed),
            pl.BlockSpec((_D, _D), fixed),
            pl.BlockSpec((1, 1), fixed),
        ],
        out_specs=pl.BlockSpec((_D, 1), fixed),
        out_shape=jax.ShapeDtypeStruct((_D, 1), jnp.float32),
        scratch_shapes=[
            pltpu.VMEM((_D, _D), jnp.float32),
            pltpu.VMEM((_D, 1), jnp.float32),
        ],
    )(lo0, lo1, hi0, hi1, h2, dinv, b2r, batch2, bng, bnb, fc1W, fc1b,
      bn1g, bn1b, fc3wr, fc3br)


def kernel(x, edge_index, batch, energy, mode, W1, b1, gcn_bn_g, gcn_bn_b,
           W2, b2, bn_g, bn_b, fc1_W, fc1_b, bn1_g, bn1_b, fc3_W, fc3_b):
    src = edge_index[0]
    dst = edge_index[1]

    x2 = jnp.pad(x, ((0, _N2 - _N), (0, 0)))
    batch2 = jnp.pad(batch, (0, _N2 - _N),
                     constant_values=_D - 1).reshape(_N2, 1)
    zeros_row = jnp.zeros((_RPT,), jnp.float32)
    zeros_blk = jnp.zeros((_ZR, _DH), jnp.float32)

    deg_parts = _sc_degree(dst, zeros_row)
    degt = deg_parts.reshape(_NC, _N2).T

    h1, hs1_lo, hs1_hi, dinv = _tc_a(x2, degt, W1)

    p1 = _sc_propagate(hs1_lo, hs1_hi, src, dst, zeros_blk)
    q1 = p1.reshape(2 * _NC, _N2, _DH)
    h2, hs2_lo, hs2_hi = _tc_b(q1[0], q1[1], q1[2], q1[3], h1, dinv,
                               b1.reshape(1, _D), gcn_bn_g.reshape(1, _D),
                               gcn_bn_b.reshape(1, _D), W2)

    p2 = _sc_propagate(hs2_lo, hs2_hi, src, dst, zeros_blk)
    q2 = p2.reshape(2 * _NC, _N2, _DH)
    fc3wr = jnp.pad(fc3_W, ((0, 0), (0, _D - 1)))
    fc3br = fc3_b.reshape(1, 1)
    out = _tc_c(q2[0], q2[1], q2[2], q2[3], h2, dinv, b2.reshape(1, _D),
                batch2, bn_g.reshape(1, _D), bn_b.reshape(1, _D), fc1_W,
                fc1_b.reshape(1, _D), bn1_g.reshape(1, _D),
                bn1_b.reshape(1, _D), fc3wr, fc3br)
    return out[:_G]
